# baseline (speedup 1.0000x reference)
import jax
import jax.numpy as jnp
import numpy as np
from jax.experimental import pallas as pl

N_NODES = 10000
N_EDGES = 320000
D = 128
MIN_EDGE_SCORE = 0.5

_BLK = 512
_GRID = N_EDGES // _BLK


def _mlp_body(cat_ref, W2_ref, b2_ref, W1_ref, b1_ref, o_ref):
    h = jnp.maximum(
        jnp.dot(cat_ref[...], W2_ref[...], preferred_element_type=jnp.float32)
        + b2_ref[...], 0.0)
    z = jnp.dot(h, W1_ref[...], preferred_element_type=jnp.float32) + b1_ref[...]
    o_ref[...] = jax.nn.sigmoid(z)


_mlp = pl.pallas_call(
    _mlp_body,
    grid=(_GRID,),
    in_specs=[pl.BlockSpec((_BLK, 2 * D), lambda i: (i, 0)),
              pl.BlockSpec((2 * D, 2 * D), lambda i: (0, 0)),
              pl.BlockSpec((1, 2 * D), lambda i: (0, 0)),
              pl.BlockSpec((2 * D, 1), lambda i: (0, 0)),
              pl.BlockSpec((1, 1), lambda i: (0, 0))],
    out_specs=pl.BlockSpec((_BLK, 1), lambda i: (i, 0)),
    out_shape=jax.ShapeDtypeStruct((N_EDGES, 1), jnp.float32),
)


def _merge(edge_score, edge_index, num_nodes, min_edge_score):
    num_edges = edge_index.shape[1]
    perm = jnp.argsort(-edge_score, stable=True)
    limit = num_edges * (1.0 - min_edge_score)
    n_iter = min(num_edges, max(0, int(np.ceil(limit)) - 1))
    src = edge_index[0]
    dst = edge_index[1]

    def body(idx, carry):
        mask, cluster, per_cluster, i = carry
        e = perm[idx]
        s = src[e]
        t = dst[e]
        ok = mask[s] & mask[t]
        cluster = cluster.at[s].set(jnp.where(ok, i, cluster[s]))
        cluster = cluster.at[t].set(jnp.where(ok, i, cluster[t]))
        mask = mask.at[s].set(jnp.where(ok, False, mask[s]))
        mask = mask.at[t].set(jnp.where(ok, False, mask[t]))
        per_cluster = per_cluster.at[i].set(
            jnp.where(ok, edge_score[e], per_cluster[i]))
        i = i + ok.astype(i.dtype)
        return (mask, cluster, per_cluster, i)

    mask0 = jnp.ones((num_nodes,), dtype=bool)
    cluster0 = -jnp.ones((num_nodes,), dtype=jnp.int32)
    per_cluster0 = jnp.ones((num_nodes,), dtype=edge_score.dtype)
    mask, cluster, per_cluster, i = jax.lax.fori_loop(
        0, n_iter, body, (mask0, cluster0, per_cluster0, jnp.int32(0)))
    cluster = jnp.where(mask, i + jnp.cumsum(mask.astype(jnp.int32)) - 1, cluster)
    return cluster, per_cluster


def kernel(x, edge_index, edge_attr, batch, W2, b2, W1, b1):
    cat = jnp.concatenate([x[edge_index[0]], x[edge_index[1]]], axis=-1)
    edge_score = _mlp(cat, W2, b2.reshape(1, -1), W1, b1.reshape(1, 1)).reshape(-1)
    cluster, per_cluster = _merge(edge_score, edge_index, N_NODES, MIN_EDGE_SCORE)
    new_x = jax.ops.segment_sum(x, cluster, num_segments=N_NODES)
    new_x = new_x * per_cluster[:, None]
    new_edge_index = cluster[edge_index]
    new_batch = jnp.zeros((N_NODES,), dtype=batch.dtype).at[cluster].set(batch)
    return (new_x, edge_score, new_edge_index, new_batch)


# speedup vs baseline: 139.9854x; 139.9854x over previous
import functools

import jax
import jax.numpy as jnp
import numpy as np
from jax import lax
from jax.experimental import pallas as pl
from jax.experimental.pallas import tpu as pltpu
from jax.experimental.pallas import tpu_sc as plsc

N_NODES = 10000
N_EDGES = 320000
D = 128
MIN_EDGE_SCORE = 0.5
N_ITER = min(N_EDGES, max(0, int(np.ceil(N_EDGES * (1.0 - MIN_EDGE_SCORE))) - 1))

_BLK = 512
_GRID = N_EDGES // _BLK


def _mlp_body(cat_ref, W2_ref, b2_ref, W1_ref, b1_ref, o_ref):
    h = jnp.maximum(
        jnp.dot(cat_ref[...], W2_ref[...], preferred_element_type=jnp.float32)
        + b2_ref[...], 0.0)
    z = jnp.dot(h, W1_ref[...], preferred_element_type=jnp.float32) + b1_ref[...]
    o_ref[...] = jax.nn.sigmoid(z)


_mlp = pl.pallas_call(
    _mlp_body,
    grid=(_GRID,),
    in_specs=[pl.BlockSpec((_BLK, 2 * D), lambda i: (i, 0)),
              pl.BlockSpec((2 * D, 2 * D), lambda i: (0, 0)),
              pl.BlockSpec((1, 2 * D), lambda i: (0, 0)),
              pl.BlockSpec((2 * D, 1), lambda i: (0, 0)),
              pl.BlockSpec((1, 1), lambda i: (0, 0))],
    out_specs=pl.BlockSpec((_BLK, 1), lambda i: (i, 0)),
    out_shape=jax.ShapeDtypeStruct((N_EDGES, 1), jnp.float32),
)


@functools.lru_cache(maxsize=None)
def _make_greedy(n_nodes, n_pad, ch, interpret=False):
    n_chunks = n_pad // ch
    mesh = plsc.VectorSubcoreMesh(core_axis_name="c", subcore_axis_name="s",
                                  num_cores=2, num_subcores=16)

    @functools.partial(
        pl.kernel,
        out_type=(jax.ShapeDtypeStruct((n_nodes,), jnp.int32),
                  jax.ShapeDtypeStruct((n_nodes,), jnp.float32),
                  jax.ShapeDtypeStruct((n_nodes,), jnp.int32),
                  jax.ShapeDtypeStruct((n_nodes,), jnp.int32)),
        mesh=mesh,
        scratch_types=[pltpu.VMEM((ch,), jnp.int32),
                       pltpu.VMEM((ch,), jnp.int32),
                       pltpu.VMEM((ch,), jnp.float32),
                       pltpu.VMEM((n_nodes,), jnp.int32),
                       pltpu.VMEM((n_nodes,), jnp.int32),
                       pltpu.VMEM((n_nodes,), jnp.float32),
                       pltpu.VMEM((n_nodes,), jnp.int32),
                       pltpu.VMEM((n_nodes,), jnp.int32)],
        compiler_params=pltpu.CompilerParams(needs_layout_passes=False),
        interpret=interpret,
    )
    def greedy(ss_hbm, dd_hbm, cc_hbm, cl_hbm, pc_hbm, m0_hbm, m1_hbm,
               ss_v, dd_v, cc_v, mask_v, cl_v, pc_v, m0_v, m1_v):
        cid = lax.axis_index("c")
        sid = lax.axis_index("s")

        @pl.when((cid == 0) & (sid == 0))
        def _():
            iota = lax.iota(jnp.int32, 16)

            def init_body(j, carry):
                b = j * 16
                mask_v[pl.ds(b, 16)] = jnp.ones((16,), jnp.int32)
                cl_v[pl.ds(b, 16)] = jnp.full((16,), -1, jnp.int32)
                pc_v[pl.ds(b, 16)] = jnp.ones((16,), jnp.float32)
                m0_v[pl.ds(b, 16)] = jnp.full((16,), n_nodes, jnp.int32)
                m1_v[pl.ds(b, 16)] = jnp.full((16,), n_nodes, jnp.int32)
                return carry

            lax.fori_loop(0, n_nodes // 16, init_body, 0)

            def chunk_body(ci, cnt):
                base = ci * ch
                pltpu.sync_copy(ss_hbm.at[pl.ds(base, ch)], ss_v)
                pltpu.sync_copy(dd_hbm.at[pl.ds(base, ch)], dd_v)
                pltpu.sync_copy(cc_hbm.at[pl.ds(base, ch)], cc_v)

                def group(g, cnt):
                    gb = g * 16
                    ss16 = ss_v[pl.ds(gb, 16)]
                    dd16 = dd_v[pl.ds(gb, 16)]
                    ms = plsc.load_gather(mask_v, [ss16])
                    mt = plsc.load_gather(mask_v, [dd16])
                    cand = ms & mt

                    def slow(cnt):
                        cc16 = cc_v[pl.ds(gb, 16)]
                        for lane in range(16):
                            def lane_fn(cnt, lane=lane):
                                s = ss16[lane]
                                t = dd16[lane]
                                sv = jnp.full((16,), s, jnp.int32)
                                tv = jnp.full((16,), t, jnp.int32)
                                idx2 = jnp.where(iota == 0, sv, tv)
                                m2 = plsc.load_gather(mask_v, [idx2])
                                ok = (m2[0] != 0) & (m2[1] != 0)

                                def acc(cnt):
                                    lanes01 = iota < 2
                                    lane0 = iota == 0
                                    plsc.store_scatter(
                                        mask_v, [idx2],
                                        jnp.zeros((16,), jnp.int32),
                                        mask=lanes01)
                                    plsc.store_scatter(
                                        cl_v, [idx2], cnt, mask=lanes01)
                                    plsc.store_scatter(
                                        pc_v, [cnt],
                                        jnp.full((16,), cc16[lane],
                                                 jnp.float32),
                                        mask=lane0)
                                    plsc.store_scatter(
                                        m0_v, [cnt], sv, mask=lane0)
                                    m1s = jnp.where(s == t, n_nodes, t)
                                    plsc.store_scatter(
                                        m1_v, [cnt],
                                        jnp.full((16,), m1s, jnp.int32),
                                        mask=lane0)
                                    return cnt + 1

                                return lax.cond(ok, acc, lambda c: c, cnt)

                            cnt = lax.cond(cand[lane] != 0, lane_fn,
                                           lambda c: c, cnt)
                        return cnt

                    return lax.cond(jnp.sum(cand) > 0, slow,
                                    lambda c: c, cnt)

                return lax.fori_loop(0, ch // 16, group, cnt)

            cnt = lax.fori_loop(0, n_chunks, chunk_body,
                                jnp.zeros((16,), jnp.int32))

            def ren_body(j, cnt):
                b = j * 16
                m = mask_v[pl.ds(b, 16)]
                mb = m != 0
                cum = plsc.cumsum(m)
                dest = cnt + cum - 1
                oldcl = cl_v[pl.ds(b, 16)]
                cl_v[pl.ds(b, 16)] = jnp.where(mb, dest, oldcl)
                plsc.store_scatter(m0_v, [dest], iota + b, mask=mb)
                return cnt + cum[15]

            lax.fori_loop(0, n_nodes // 16, ren_body, cnt)

            pltpu.sync_copy(cl_v, cl_hbm)
            pltpu.sync_copy(pc_v, pc_hbm)
            pltpu.sync_copy(m0_v, m0_hbm)
            pltpu.sync_copy(m1_v, m1_hbm)

    return greedy


_SLAB = 64


@functools.lru_cache(maxsize=None)
def _make_pool(n_nodes, n_edges, d):
    mesh = plsc.VectorSubcoreMesh(core_axis_name="c", subcore_axis_name="s",
                                  num_cores=2, num_subcores=16)
    eids_w = (2 * n_edges) // 32
    n_full = n_nodes // _SLAB
    n_tail = n_nodes - n_full * _SLAB

    @functools.partial(
        pl.kernel,
        out_type=(jax.ShapeDtypeStruct((n_nodes, d), jnp.float32),
                  jax.ShapeDtypeStruct((2 * n_edges,), jnp.int32),
                  jax.ShapeDtypeStruct((n_nodes,), jnp.int32)),
        mesh=mesh,
        scratch_types=[pltpu.VMEM((_SLAB, d), jnp.float32),
                       pltpu.VMEM((_SLAB, d), jnp.float32),
                       pltpu.VMEM((_SLAB,), jnp.int32),
                       pltpu.VMEM((_SLAB,), jnp.int32),
                       pltpu.VMEM((_SLAB,), jnp.float32),
                       pltpu.VMEM((n_nodes,), jnp.int32),
                       pltpu.VMEM((eids_w,), jnp.int32),
                       pltpu.VMEM((n_nodes + 16,), jnp.int32),
                       pltpu.SemaphoreType.DMA,
                       pltpu.SemaphoreType.DMA],
        compiler_params=pltpu.CompilerParams(needs_layout_passes=False),
    )
    def pool(xext_hbm, cl_hbm, pc_hbm, m0_hbm, m1_hbm, bext_hbm, ei_hbm,
             nx_hbm, nei_hbm, nb_hbm,
             r0_v, r1_v, m0_v, m1_v, pc_v, cl_v, ei_v, b_v, sem0, sem1):
        cid = lax.axis_index("c")
        sid = lax.axis_index("s")
        wid = sid * 2 + cid

        pltpu.sync_copy(bext_hbm, b_v.at[pl.ds(0, n_nodes + 1)])

        def slab(b, nr):
            pltpu.sync_copy(m0_hbm.at[pl.ds(b, nr)], m0_v.at[pl.ds(0, nr)])
            pltpu.sync_copy(m1_hbm.at[pl.ds(b, nr)], m1_v.at[pl.ds(0, nr)])
            pltpu.sync_copy(pc_hbm.at[pl.ds(b, nr)], pc_v.at[pl.ds(0, nr)])
            cp0 = pltpu.async_copy(
                xext_hbm.at[m0_v.at[pl.ds(0, nr)]], r0_v.at[pl.ds(0, nr)],
                sem0)
            cp1 = pltpu.async_copy(
                xext_hbm.at[m1_v.at[pl.ds(0, nr)]], r1_v.at[pl.ds(0, nr)],
                sem1)
            cp0.wait()
            cp1.wait()

            def row_body(r, carry):
                pcs = plsc.load_gather(pc_v, [jnp.full((16,), r, jnp.int32)])

                def col_body(cb, carry2):
                    col = cb * 16
                    r0_v[r, pl.ds(col, 16)] = (
                        (r0_v[r, pl.ds(col, 16)] + r1_v[r, pl.ds(col, 16)])
                        * pcs)
                    return carry2

                lax.fori_loop(0, d // 16, col_body, 0)
                return carry

            lax.fori_loop(0, nr, row_body, 0)
            pltpu.sync_copy(r0_v.at[pl.ds(0, nr)], nx_hbm.at[pl.ds(b, nr)])

            def nb_body(g, carry):
                gb = g * 16
                i0 = m0_v[pl.ds(gb, 16)]
                i1 = m1_v[pl.ds(gb, 16)]
                v0 = plsc.load_gather(b_v, [i0])
                v1 = plsc.load_gather(b_v, [i1])
                m0_v[pl.ds(gb, 16)] = jnp.maximum(v0, v1)
                return carry

            lax.fori_loop(0, nr // 16, nb_body, 0)
            pltpu.sync_copy(m0_v.at[pl.ds(0, nr)], nb_hbm.at[pl.ds(b, nr)])

        def slab_loop(g, carry):
            sl = g * 32 + wid

            @pl.when(sl < n_full)
            def _():
                slab(pl.multiple_of(sl * _SLAB, _SLAB), _SLAB)

            return carry

        lax.fori_loop(0, (n_full + 31) // 32, slab_loop, 0)

        if n_tail:
            @pl.when(wid == 0)
            def _():
                slab(n_full * _SLAB, n_tail)

        pltpu.sync_copy(cl_hbm, cl_v)
        off = pl.multiple_of(wid * eids_w, eids_w)
        pltpu.sync_copy(ei_hbm.at[pl.ds(off, eids_w)], ei_v)

        def remap(g, carry):
            gb = g * 16
            ids = ei_v[pl.ds(gb, 16)]
            ei_v[pl.ds(gb, 16)] = plsc.load_gather(cl_v, [ids])
            return carry

        lax.fori_loop(0, eids_w // 16, remap, 0)
        pltpu.sync_copy(ei_v, nei_hbm.at[pl.ds(off, eids_w)])

    return pool


_SCB = 1


@functools.lru_cache(maxsize=None)
def _make_sort(n_edges):
    mesh = plsc.VectorSubcoreMesh(core_axis_name="c", subcore_axis_name="s",
                                  num_cores=1, num_subcores=16)
    W = n_edges // 16
    NFR = W // 128
    REMG = (W - NFR * 128) // 16

    @functools.partial(
        pl.kernel,
        out_type=(jax.ShapeDtypeStruct((n_edges,), jnp.int32),
                  jax.ShapeDtypeStruct((n_edges,), jnp.int32),
                  jax.ShapeDtypeStruct((n_edges,), jnp.float32),
                  jax.ShapeDtypeStruct((n_edges,), jnp.int32),
                  jax.ShapeDtypeStruct((n_edges,), jnp.int32),
                  jax.ShapeDtypeStruct((n_edges,), jnp.int32),
                  jax.ShapeDtypeStruct((n_edges,), jnp.int32)),
        mesh=mesh,
        scratch_types=[pltpu.VMEM((W,), jnp.int32),
                       pltpu.VMEM((W,), jnp.int32),
                       pltpu.VMEM((W,), jnp.float32),
                       pltpu.VMEM((NFR + 1, 128), jnp.int32),
                       pltpu.VMEM((4096,), jnp.int32),
                       pltpu.VMEM((256,), jnp.int32),
                       pltpu.VMEM((4096,), jnp.int32),
                       pltpu.VMEM((256,), jnp.int32),
                       pltpu.VMEM_SHARED((4096,), jnp.int32),
                       pltpu.SemaphoreType.DMA,
                       pltpu.SemaphoreType.DMA,
                       pltpu.SemaphoreType.DMA],
        compiler_params=pltpu.CompilerParams(needs_layout_passes=False),
    )
    def rsort(score_hbm, src_hbm, dst_hbm, ss_hbm, dd_hbm, cc_hbm,
              kb0, kb1, vb0, vb1,
              kw_v, vw_v, fw_v, dest2d, hist16, histmine, histall, mybase,
              hist_sh, sem0, sem1, sem2):
        sid = lax.axis_index("s")
        iota = lax.iota(jnp.int32, 16)
        base = pl.multiple_of(sid * W, W)
        minint = jnp.int32(-2**31)

        pltpu.sync_copy(score_hbm.at[pl.ds(base, W)], fw_v)

        def stage(g, carry):
            o = g * 16
            m = plsc.bitcast(fw_v[pl.ds(o, 16)], jnp.int32)
            sign = lax.shift_right_arithmetic(m, 31)
            key = m ^ (sign | minint) ^ jnp.int32(-1)
            kw_v[pl.ds(o, 16)] = key
            vw_v[pl.ds(o, 16)] = iota + (base + o)
            return carry

        lax.fori_loop(0, W // 16, stage, 0)
        pltpu.sync_copy(kw_v, kb0.at[pl.ds(base, W)])
        pltpu.sync_copy(vw_v, vb0.at[pl.ds(base, W)])
        plsc.subcore_barrier()

        for p in range(4):
            rk, rv, wk, wv = ((kb0, vb0, kb1, vb1) if p % 2 == 0
                              else (kb1, vb1, kb0, vb0))
            sh = 8 * p
            pltpu.sync_copy(rk.at[pl.ds(base, W)], kw_v)
            pltpu.sync_copy(rv.at[pl.ds(base, W)], vw_v)

            def zero(g, carry):
                hist16[pl.ds(g * 16, 16)] = jnp.zeros((16,), jnp.int32)
                return carry

            lax.fori_loop(0, 256, zero, 0)

            def hist(g, carry, sh=sh):
                k16 = kw_v[pl.ds(g * 16, 16)]
                dig = lax.shift_right_logical(k16, sh) & 255
                plsc.addupdate_scatter(hist16, [iota * 256 + dig],
                                       jnp.ones((16,), jnp.int32))
                return carry

            lax.fori_loop(0, W // 16, hist, 0)

            for g in range(16):
                acc = jnp.zeros((16,), jnp.int32)
                for r in range(16):
                    acc = acc + hist16[pl.ds(r * 256 + g * 16, 16)]
                histmine[pl.ds(g * 16, 16)] = acc
            pltpu.sync_copy(histmine,
                            hist_sh.at[pl.ds(pl.multiple_of(sid * 256, 256),
                                             256)])
            plsc.subcore_barrier()
            pltpu.sync_copy(hist_sh, histall)

            carry = jnp.int32(0)
            for g in range(16):
                tot = jnp.zeros((16,), jnp.int32)
                part = jnp.zeros((16,), jnp.int32)
                for t in range(16):
                    row = histall[pl.ds(t * 256 + g * 16, 16)]
                    tot = tot + row
                    part = part + jnp.where(
                        jnp.full((16,), t, jnp.int32) < sid, row, 0)
                pc = plsc.cumsum(tot)
                mybase[pl.ds(g * 16, 16)] = pc - tot + carry + part
                carry = carry + pc[15]

            def permute_group(o, sh=sh):
                k16 = kw_v[pl.ds(o, 16)]
                dig = lax.shift_right_logical(k16, sh) & 255
                b16 = plsc.load_gather(mybase, [dig])
                cnt, lastm = plsc.scan_count(dig)
                dest = b16 + cnt - _SCB
                plsc.store_scatter(mybase, [dig], dest + 1, mask=lastm)
                return dest

            def prow(j, carry2):
                off = pl.multiple_of(j * 128, 128)
                for gg in range(8):
                    dest2d[j, pl.ds(gg * 16, 16)] = permute_group(
                        off + gg * 16)
                pltpu.sync_copy(kw_v.at[pl.ds(off, 128)],
                                wk.at[dest2d.at[j]])
                pltpu.sync_copy(vw_v.at[pl.ds(off, 128)],
                                wv.at[dest2d.at[j]])
                return carry2

            lax.fori_loop(0, NFR, prow, 0)
            if REMG:
                roff = NFR * 128
                for gg in range(REMG):
                    dest2d[NFR, pl.ds(gg * 16, 16)] = permute_group(
                        roff + gg * 16)
                pltpu.sync_copy(kw_v.at[pl.ds(roff, REMG * 16)],
                                wk.at[dest2d.at[NFR, pl.ds(0, REMG * 16)]])
                pltpu.sync_copy(vw_v.at[pl.ds(roff, REMG * 16)],
                                wv.at[dest2d.at[NFR, pl.ds(0, REMG * 16)]])
            plsc.subcore_barrier()

        pltpu.sync_copy(vb0.at[pl.ds(base, W)], vw_v)

        def grow(j, carry2):
            off = pl.multiple_of(j * 128, 128)
            idx = vw_v.at[pl.ds(off, 128)]
            c0 = pltpu.async_copy(src_hbm.at[idx],
                                  kw_v.at[pl.ds(off, 128)], sem0)
            c1 = pltpu.async_copy(dst_hbm.at[idx],
                                  dest2d.at[j], sem1)
            c2 = pltpu.async_copy(score_hbm.at[idx],
                                  fw_v.at[pl.ds(off, 128)], sem2)
            c0.wait()
            c1.wait()
            c2.wait()
            return carry2

        lax.fori_loop(0, NFR, grow, 0)
        if REMG:
            roff = NFR * 128
            n = REMG * 16
            idx = vw_v.at[pl.ds(roff, n)]
            c0 = pltpu.async_copy(src_hbm.at[idx],
                                  kw_v.at[pl.ds(roff, n)], sem0)
            c1 = pltpu.async_copy(dst_hbm.at[idx],
                                  dest2d.at[NFR, pl.ds(0, n)], sem1)
            c2 = pltpu.async_copy(score_hbm.at[idx],
                                  fw_v.at[pl.ds(roff, n)], sem2)
            c0.wait()
            c1.wait()
            c2.wait()
        pltpu.sync_copy(kw_v, ss_hbm.at[pl.ds(base, W)])
        pltpu.sync_copy(fw_v, cc_hbm.at[pl.ds(base, W)])

        def dout(j, carry2):
            off = pl.multiple_of(j * 128, 128)
            pltpu.sync_copy(dest2d.at[j],
                            dd_hbm.at[pl.ds(base + off, 128)])
            return carry2

        lax.fori_loop(0, NFR, dout, 0)
        if REMG:
            roff = NFR * 128
            pltpu.sync_copy(dest2d.at[NFR, pl.ds(0, REMG * 16)],
                            dd_hbm.at[pl.ds(base + roff, REMG * 16)])

    return rsort


_GCH = 128


@functools.lru_cache(maxsize=None)
def _make_catgather(n_nodes, n_edges, d):
    mesh = plsc.VectorSubcoreMesh(core_axis_name="c", subcore_axis_name="s",
                                  num_cores=2, num_subcores=16)
    rows_w = n_edges // 32
    n_ch = rows_w // _GCH
    rem = rows_w - n_ch * _GCH

    @functools.partial(
        pl.kernel,
        out_type=jax.ShapeDtypeStruct((n_edges, 2 * d), jnp.float32),
        mesh=mesh,
        scratch_types=[pltpu.VMEM((rows_w,), jnp.int32),
                       pltpu.VMEM((rows_w,), jnp.int32),
                       pltpu.VMEM((_GCH, d), jnp.float32),
                       pltpu.VMEM((_GCH, d), jnp.float32),
                       pltpu.SemaphoreType.DMA,
                       pltpu.SemaphoreType.DMA],
        compiler_params=pltpu.CompilerParams(needs_layout_passes=False),
    )
    def catgather(x_hbm, src_hbm, dst_hbm, cat_hbm,
                  si_v, di_v, r0_v, r1_v, sem0, sem1):
        cid = lax.axis_index("c")
        sid = lax.axis_index("s")
        wid = sid * 2 + cid
        base = pl.multiple_of(wid * rows_w, rows_w)
        pltpu.sync_copy(src_hbm.at[pl.ds(base, rows_w)], si_v)
        pltpu.sync_copy(dst_hbm.at[pl.ds(base, rows_w)], di_v)

        def chunk(c, nr):
            off = pl.multiple_of(c * _GCH, _GCH)
            cp0 = pltpu.async_copy(
                x_hbm.at[si_v.at[pl.ds(off, nr)]], r0_v.at[pl.ds(0, nr)],
                sem0)
            cp1 = pltpu.async_copy(
                x_hbm.at[di_v.at[pl.ds(off, nr)]], r1_v.at[pl.ds(0, nr)],
                sem1)
            cp0.wait()
            cp1.wait()
            pltpu.sync_copy(
                r0_v.at[pl.ds(0, nr)],
                cat_hbm.at[pl.ds(base + off, nr), pl.ds(0, d)])
            pltpu.sync_copy(
                r1_v.at[pl.ds(0, nr)],
                cat_hbm.at[pl.ds(base + off, nr), pl.ds(d, d)])

        def chunk_loop(c, carry):
            chunk(c, _GCH)
            return carry

        lax.fori_loop(0, n_ch, chunk_loop, 0)
        if rem:
            chunk(n_ch, rem)

    return catgather


def kernel(x, edge_index, edge_attr, batch, W2, b2, W1, b1):
    cat = jnp.concatenate([x[edge_index[0]], x[edge_index[1]]], axis=-1)
    edge_score = _mlp(cat, W2, b2.reshape(1, -1), W1, b1.reshape(1, 1)).reshape(-1)

    perm = jnp.argsort(-edge_score, stable=True)
    top = perm[:N_ITER]
    top = jnp.concatenate([top, top[N_ITER - 1:N_ITER]])
    ss = edge_index[0][top]
    dd = edge_index[1][top]
    cc = edge_score[top]

    greedy = _make_greedy(N_NODES, N_ITER + 1, (N_ITER + 1) // 20)
    cluster, per_cluster, m0, m1 = greedy(ss, dd, cc)

    x_ext = jnp.concatenate([x, jnp.zeros((1, D), x.dtype)])
    batch_ext = jnp.concatenate([batch, jnp.zeros((1,), batch.dtype)])
    pool = _make_pool(N_NODES, N_EDGES, D)
    new_x, new_ei_flat, new_batch = pool(
        x_ext, cluster, per_cluster, m0, m1, batch_ext,
        edge_index.reshape(-1))
    new_edge_index = new_ei_flat.reshape(2, N_EDGES)
    return (new_x, edge_score, new_edge_index, new_batch)
